# baseline (speedup 1.0000x reference)
import functools

import jax
import jax.numpy as jnp
from jax import lax
from jax.experimental import pallas as pl
from jax.experimental.pallas import tpu as pltpu
from jax.experimental.pallas import tpu_sc as plsc

_NC, _NS = 2, 16
_NW = _NC * _NS
_ROWS, _D = 4096, 1024
_RPW = _ROWS // _NW
_CH = 16
_NCH = _RPW // _CH
_NBUF = 4

_mesh = plsc.VectorSubcoreMesh(core_axis_name="c", subcore_axis_name="s")


@functools.partial(
    pl.kernel,
    out_type=jax.ShapeDtypeStruct((_ROWS, _D), jnp.float32),
    mesh=_mesh,
    scratch_types=(
        [pltpu.VMEM((_CH, _D), jnp.float32)] * _NBUF
        + [pltpu.SemaphoreType.DMA] * _NBUF
        + [pltpu.SemaphoreType.DMA] * _NBUF
    ),
)
def _sc_copy(x_hbm, o_hbm, *scr):
    bufs = scr[:_NBUF]
    lsems = scr[_NBUF:2 * _NBUF]
    ssems = scr[2 * _NBUF:]
    wid = lax.axis_index("s") * _NC + lax.axis_index("c")
    base = wid * _RPW

    loads = {}
    stores = {}
    for i in range(min(_NBUF, _NCH)):
        loads[i] = pltpu.async_copy(
            x_hbm.at[pl.ds(base + i * _CH, _CH), :], bufs[i], lsems[i]
        )
    for i in range(_NCH):
        b = i % _NBUF
        loads[i].wait()
        stores[i] = pltpu.async_copy(
            bufs[b], o_hbm.at[pl.ds(base + i * _CH, _CH), :], ssems[b]
        )
        j = i + _NBUF
        if j < _NCH:
            stores[i].wait()
            loads[j] = pltpu.async_copy(
                x_hbm.at[pl.ds(base + j * _CH, _CH), :], bufs[b], lsems[b]
            )
    for i in range(max(0, _NCH - _NBUF), _NCH):
        stores[i].wait()


def kernel(x):
    b, s, d = x.shape
    out = _sc_copy(x.reshape(_ROWS, _D))
    return out.reshape(b, s, d)


# speedup vs baseline: 2.7998x; 2.7998x over previous
import jax
import jax.numpy as jnp
from jax.experimental import pallas as pl
from jax.experimental.pallas import tpu as pltpu

_ROWS = 4096
_D = 1024
_CHUNKS = (512, 3072, 512)
_OFFS = tuple(sum(_CHUNKS[:i]) for i in range(len(_CHUNKS)))
_N = len(_CHUNKS)


def _copy_body(x_ref, o_ref, vmem, load_sems, store_sems):
    loads = []
    for i in range(_N):
        c = pltpu.make_async_copy(
            x_ref.at[pl.ds(_OFFS[i], _CHUNKS[i]), :],
            vmem.at[pl.ds(_OFFS[i], _CHUNKS[i]), :],
            load_sems.at[i],
        )
        c.start()
        loads.append(c)
    stores = []
    for i in range(_N):
        loads[i].wait()
        c = pltpu.make_async_copy(
            vmem.at[pl.ds(_OFFS[i], _CHUNKS[i]), :],
            o_ref.at[pl.ds(_OFFS[i], _CHUNKS[i]), :],
            store_sems.at[i],
        )
        c.start()
        stores.append(c)
    for c in stores:
        c.wait()


def kernel(x):
    b, s, d = x.shape
    x2 = x.reshape(_ROWS, _D)
    out = pl.pallas_call(
        _copy_body,
        in_specs=[pl.BlockSpec(memory_space=pl.ANY)],
        out_specs=pl.BlockSpec(memory_space=pl.ANY),
        scratch_shapes=[
            pltpu.VMEM((_ROWS, _D), jnp.float32),
            pltpu.SemaphoreType.DMA((_N,)),
            pltpu.SemaphoreType.DMA((_N,)),
        ],
        out_shape=jax.ShapeDtypeStruct((_ROWS, _D), x.dtype),
    )(x2)
    return out.reshape(b, s, d)
